# baseline (speedup 1.0000x reference)
import functools

import jax
import jax.numpy as jnp
from jax import lax
from jax.experimental import pallas as pl
from jax.experimental.pallas import tpu as pltpu
from jax.experimental.pallas import tpu_sc as plsc

E = 8
D = 2048
N = 32768
BLOCK_N = 1024
L = 16


def _affine_body(x_ref, g_ref, b_ref, out_ref):
    out_ref[...] = x_ref[...] * g_ref[...] + b_ref[...]


_SC_MESH = plsc.VectorSubcoreMesh(core_axis_name="c", subcore_axis_name="s")


@functools.partial(
    pl.kernel,
    mesh=_SC_MESH,
    out_type=jax.ShapeDtypeStruct((L,), jnp.float32),
    scratch_types=[
        pltpu.VMEM((2 * L,), jnp.float32),
        pltpu.VMEM((L,), jnp.float32),
        pltpu.VMEM((L,), jnp.float32),
        pltpu.VMEM((L,), jnp.float32),
    ],
)
def _gate_sc(packed_hbm, lat_hbm, in_v, e_v, we_v, out_v):
    wid = lax.axis_index("s") * 2 + lax.axis_index("c")

    @pl.when(wid == 0)
    def _():
        pltpu.sync_copy(packed_hbm, in_v)
        a = in_v[pl.ds(0, L)]
        lat0 = in_v[pl.ds(L, L)]
        e = jnp.exp(a)
        i = lax.broadcasted_iota(jnp.int32, (L,), 0).astype(jnp.float32)
        w = 0.5 + i * (1.5 / (E - 1))
        we = w * e
        num = we[0]
        den = e[0]
        for j in range(1, E):
            num = num + we[j]
            den = den + e[j]
        numv = jnp.full((L,), num, jnp.float32)
        denv = jnp.full((L,), den, jnp.float32)
        out_v[...] = lat0 + numv / denv
        pltpu.sync_copy(out_v, lat_hbm)


@jax.jit
def kernel(x, latency_to_accumulate, AP_path_alpha, gammas, betas):
    packed = jnp.concatenate(
        [
            AP_path_alpha,
            jnp.full((L - E,), -jnp.inf, jnp.float32),
            jnp.full((L,), latency_to_accumulate, jnp.float32),
        ]
    )
    lat_vec = _gate_sc(packed)

    grid = (N // BLOCK_N,)
    out = pl.pallas_call(
        _affine_body,
        grid=grid,
        in_specs=[
            pl.BlockSpec((BLOCK_N, D), lambda i: (i, 0)),
            pl.BlockSpec((1, D), lambda i: (0, 0)),
            pl.BlockSpec((1, D), lambda i: (0, 0)),
        ],
        out_specs=pl.BlockSpec((BLOCK_N, D), lambda i: (i, 0)),
        out_shape=jax.ShapeDtypeStruct((N, D), jnp.float32),
    )(x, gammas[0:1], betas[0:1])
    return out, lat_vec[0]


# speedup vs baseline: 1.0200x; 1.0200x over previous
import functools

import jax
import jax.numpy as jnp
from jax import lax
from jax.experimental import pallas as pl
from jax.experimental.pallas import tpu as pltpu
from jax.experimental.pallas import tpu_sc as plsc

E = 8
D = 2048
N = 32768
BLOCK_N = 1024
L = 16


def _affine_body(x_ref, g_ref, b_ref, out_ref):
    out_ref[...] = x_ref[...] * g_ref[0:1, :] + b_ref[0:1, :]


_SC_MESH = plsc.VectorSubcoreMesh(core_axis_name="c", subcore_axis_name="s")


@functools.partial(
    pl.kernel,
    mesh=_SC_MESH,
    out_type=jax.ShapeDtypeStruct((L,), jnp.float32),
    scratch_types=[
        pltpu.VMEM((L,), jnp.float32),
        pltpu.VMEM((L,), jnp.float32),
        pltpu.VMEM((L,), jnp.float32),
    ],
)
def _gate_sc(alpha_hbm, lat0_hbm, lat_hbm, in_v, lat0_v, out_v):
    wid = lax.axis_index("s") * 2 + lax.axis_index("c")

    @pl.when(wid == 0)
    def _():
        in_v[...] = jnp.full((L,), -jnp.inf, jnp.float32)
        pltpu.sync_copy(alpha_hbm, in_v.at[pl.ds(0, E)])
        pltpu.sync_copy(lat0_hbm, lat0_v)
        a = in_v[...]
        e = jnp.exp(a)
        i = lax.broadcasted_iota(jnp.int32, (L,), 0).astype(jnp.float32)
        w = 0.5 + i * (1.5 / (E - 1))
        we = w * e
        num = we[0]
        den = e[0]
        for j in range(1, E):
            num = num + we[j]
            den = den + e[j]
        numv = jnp.full((L,), num, jnp.float32)
        denv = jnp.full((L,), den, jnp.float32)
        out_v[...] = lat0_v[...] + numv / denv
        pltpu.sync_copy(out_v, lat_hbm)


@jax.jit
def kernel(x, latency_to_accumulate, AP_path_alpha, gammas, betas):
    lat0_vec = jnp.full((L,), latency_to_accumulate, jnp.float32)
    lat_vec = _gate_sc(AP_path_alpha, lat0_vec)

    grid = (N // BLOCK_N,)
    out = pl.pallas_call(
        _affine_body,
        grid=grid,
        in_specs=[
            pl.BlockSpec((BLOCK_N, D), lambda i: (i, 0)),
            pl.BlockSpec((E, D), lambda i: (0, 0)),
            pl.BlockSpec((E, D), lambda i: (0, 0)),
        ],
        out_specs=pl.BlockSpec((BLOCK_N, D), lambda i: (i, 0)),
        out_shape=jax.ShapeDtypeStruct((N, D), jnp.float32),
    )(x, gammas, betas)
    return out, lat_vec[0]


# speedup vs baseline: 1.1140x; 1.0922x over previous
import jax
import jax.numpy as jnp
from jax import lax
from jax.experimental import pallas as pl

E = 8
D = 2048
N = 32768
BLOCK_N = 1024


def _affine_body(x_ref, lat0_ref, alpha_ref, g_ref, b_ref, out_ref, lat_ref):
    out_ref[...] = x_ref[...] * g_ref[0:1, :] + b_ref[0:1, :]

    @pl.when(pl.program_id(0) == 0)
    def _():
        a = alpha_ref[...]
        m = jnp.max(a)
        e = jnp.exp(a - m)
        i = lax.broadcasted_iota(jnp.int32, (1, E), 1).astype(jnp.float32)
        latency = 0.5 + i * (1.5 / (E - 1))
        lat_ref[...] = lat0_ref[...] + jnp.sum(latency * e) / jnp.sum(e)


@jax.jit
def kernel(x, latency_to_accumulate, AP_path_alpha, gammas, betas):
    grid = (N // BLOCK_N,)
    out, lat = pl.pallas_call(
        _affine_body,
        grid=grid,
        in_specs=[
            pl.BlockSpec((BLOCK_N, D), lambda i: (i, 0)),
            pl.BlockSpec((1, 1), lambda i: (0, 0)),
            pl.BlockSpec((1, E), lambda i: (0, 0)),
            pl.BlockSpec((E, D), lambda i: (0, 0)),
            pl.BlockSpec((E, D), lambda i: (0, 0)),
        ],
        out_specs=[
            pl.BlockSpec((BLOCK_N, D), lambda i: (i, 0)),
            pl.BlockSpec((1, 1), lambda i: (0, 0)),
        ],
        out_shape=[
            jax.ShapeDtypeStruct((N, D), jnp.float32),
            jax.ShapeDtypeStruct((1, 1), jnp.float32),
        ],
    )(
        x,
        latency_to_accumulate.reshape(1, 1),
        AP_path_alpha.reshape(1, E),
        gammas,
        betas,
    )
    return out, lat.reshape(())


# speedup vs baseline: 1.1143x; 1.0002x over previous
import jax
import jax.numpy as jnp
from jax import lax
from jax.experimental import pallas as pl
from jax.experimental.pallas import tpu as pltpu

E = 8
D = 2048
N = 32768
BLOCK_N = 1024


def _affine_body(x_ref, lat0_ref, alpha_ref, g_ref, b_ref, out_ref, lat_ref):
    out_ref[...] = x_ref[...] * g_ref[0:1, :] + b_ref[0:1, :]

    @pl.when(pl.program_id(0) == 0)
    def _():
        a = alpha_ref[...]
        m = jnp.max(a)
        e = jnp.exp(a - m)
        i = lax.broadcasted_iota(jnp.int32, (1, E), 1).astype(jnp.float32)
        latency = 0.5 + i * (1.5 / (E - 1))
        lat_ref[...] = lat0_ref[...] + jnp.sum(latency * e) / jnp.sum(e)


@jax.jit
def kernel(x, latency_to_accumulate, AP_path_alpha, gammas, betas):
    grid = (N // BLOCK_N,)
    out, lat = pl.pallas_call(
        _affine_body,
        grid=grid,
        compiler_params=pltpu.CompilerParams(
            dimension_semantics=("parallel",),
        ),
        in_specs=[
            pl.BlockSpec((BLOCK_N, D), lambda i: (i, 0)),
            pl.BlockSpec((1, 1), lambda i: (0, 0)),
            pl.BlockSpec((1, E), lambda i: (0, 0)),
            pl.BlockSpec((E, D), lambda i: (0, 0)),
            pl.BlockSpec((E, D), lambda i: (0, 0)),
        ],
        out_specs=[
            pl.BlockSpec((BLOCK_N, D), lambda i: (i, 0)),
            pl.BlockSpec((1, 1), lambda i: (0, 0)),
        ],
        out_shape=[
            jax.ShapeDtypeStruct((N, D), jnp.float32),
            jax.ShapeDtypeStruct((1, 1), jnp.float32),
        ],
    )(
        x,
        latency_to_accumulate.reshape(1, 1),
        AP_path_alpha.reshape(1, E),
        gammas,
        betas,
    )
    return out, lat.reshape(())


# speedup vs baseline: 1.1149x; 1.0005x over previous
import jax
import jax.numpy as jnp
from jax import lax
from jax.experimental import pallas as pl

E = 8
D = 2048
N = 32768
BLOCK_N = 1024


def _affine_body(x_ref, lat0_ref, alpha_ref, g_ref, b_ref, out_ref, lat_ref):
    out_ref[...] = x_ref[...] * g_ref[0:1, :] + b_ref[0:1, :]

    @pl.when(pl.program_id(0) == 0)
    def _():
        a = alpha_ref[...]
        m = jnp.max(a)
        e = jnp.exp(a - m)
        i = lax.broadcasted_iota(jnp.int32, (1, E), 1).astype(jnp.float32)
        latency = 0.5 + i * (1.5 / (E - 1))
        lat_ref[...] = lat0_ref[...] + jnp.sum(latency * e) / jnp.sum(e)


@jax.jit
def kernel(x, latency_to_accumulate, AP_path_alpha, gammas, betas):
    grid = (N // BLOCK_N,)
    out, lat = pl.pallas_call(
        _affine_body,
        grid=grid,
        in_specs=[
            pl.BlockSpec((BLOCK_N, D), lambda i: (i, 0)),
            pl.BlockSpec((1, 1), lambda i: (0, 0)),
            pl.BlockSpec((1, E), lambda i: (0, 0)),
            pl.BlockSpec((E, D), lambda i: (0, 0)),
            pl.BlockSpec((E, D), lambda i: (0, 0)),
        ],
        out_specs=[
            pl.BlockSpec((BLOCK_N, D), lambda i: (i, 0)),
            pl.BlockSpec((1, 1), lambda i: (0, 0)),
        ],
        out_shape=[
            jax.ShapeDtypeStruct((N, D), jnp.float32),
            jax.ShapeDtypeStruct((1, 1), jnp.float32),
        ],
    )(
        x,
        latency_to_accumulate.reshape(1, 1),
        AP_path_alpha.reshape(1, E),
        gammas,
        betas,
    )
    return out, lat.reshape(())
